# baseline (speedup 1.0000x reference)
import functools
import math

import jax
import jax.numpy as jnp
from jax import lax
from jax.experimental import pallas as pl
from jax.experimental.pallas import tpu as pltpu
from jax.experimental.pallas import tpu_sc as plsc

N = 2000000
G = 1024
GG = G * G
HALF = 2 * GG
PAD = 16384
OUTW = HALF + PAD
DUMP = HALF
GDUMP = 4 * GG

B = 16000
NB = N // B

NTILE = 16
ROWS_PT = 984
NPAD = NTILE * ROWS_PT * 128
CHUNK = 24
NCHUNK = ROWS_PT // CHUNK

BR = 64
TANH1 = math.tanh(1.0)
VSCALE = 2.0 / math.sqrt(2.0)


def _prep_body(team_ref, px, py, ty, tm, f0, t0_ref, t1_ref, es_ref, ec_ref):
    i = pl.program_id(0)
    x = px[0, 0, :]
    y = py[0, 0, :]
    t = ty[0, 0, :]
    m = tm[0, 0, :]
    team = team_ref[0, 0]
    mx = jnp.clip((x * float(G)).astype(jnp.int32), 0, G - 1)
    my = jnp.clip((y * float(G)).astype(jnp.int32), 0, G - 1)
    flat = my * G + mx
    is_obstacle = t == 3
    is_ally = (t == 0) & (m == team)
    is_enemy = (t == 0) & (m != team) & (m >= 0)
    is_resource = t == 1
    ch = jnp.where(
        is_obstacle, 0,
        jnp.where(is_ally, 1, jnp.where(is_enemy, 2, jnp.where(is_resource, 3, -1))))
    valid = ch >= 0
    flat4 = jnp.where(valid, flat + GG * jnp.maximum(ch, 0), GDUMP)
    t0 = jnp.where(flat4 < HALF, flat4, DUMP)
    t1 = jnp.where((flat4 >= HALF) & (flat4 < 4 * GG),
                   flat4 - HALF + OUTW, DUMP + OUTW)
    t0_ref[0, 0, :] = t0
    t1_ref[0, 0, :] = t1
    ally_f = is_ally.astype(jnp.float32)
    bs = jnp.sum(f0[0, 0, :] * ally_f)
    bc = jnp.sum(ally_f)

    @pl.when(i == 0)
    def _():
        es_ref[0, 0] = 0.0
        ec_ref[0, 0] = 0.0

    es_ref[0, 0] += bs
    ec_ref[0, 0] += bc


def _prep(px, py, ty, tm, f0, team):
    spec = lambda dt: pl.BlockSpec((1, 1, B), lambda i: (i, 0, 0))
    sspec = pl.BlockSpec((1, 1), lambda i: (0, 0), memory_space=pltpu.SMEM)
    return pl.pallas_call(
        _prep_body,
        grid=(NB,),
        in_specs=[sspec, spec(None), spec(None), spec(None), spec(None), spec(None)],
        out_specs=[
            pl.BlockSpec((1, 1, B), lambda i: (i, 0, 0)),
            pl.BlockSpec((1, 1, B), lambda i: (i, 0, 0)),
            sspec, sspec,
        ],
        out_shape=[
            jax.ShapeDtypeStruct((NB, 1, B), jnp.int32),
            jax.ShapeDtypeStruct((NB, 1, B), jnp.int32),
            jax.ShapeDtypeStruct((1, 1), jnp.float32),
            jax.ShapeDtypeStruct((1, 1), jnp.float32),
        ],
    )(team, px, py, ty, tm, f0)


def _sc_scatter(tgt):
    mesh = plsc.VectorSubcoreMesh(core_axis_name="c", subcore_axis_name="s")

    @functools.partial(
        pl.kernel,
        mesh=mesh,
        out_type=jax.ShapeDtypeStruct((2 * OUTW,), jnp.float32),
        scratch_types=[
            pltpu.VMEM((CHUNK, 128), jnp.int32),
            pltpu.VMEM((128,), jnp.float32),
            pltpu.VMEM((8256,), jnp.float32),
            pltpu.SemaphoreType.DMA,
        ],
    )
    def k(tgt_hbm, out_hbm, stg_v, ones_v, zero_v, sem):
        c = lax.axis_index("c")
        s = lax.axis_index("s")

        def fill_ones(j, _):
            ones_v[pl.ds(j * 16, 16)] = jnp.ones((16,), jnp.float32)
            return _

        lax.fori_loop(0, 8, fill_ones, 0, unroll=True)

        def fill_zero(j, _):
            zero_v[pl.ds(j * 16, 16)] = jnp.zeros((16,), jnp.float32)
            return _

        lax.fori_loop(0, 516, fill_zero, 0)

        cells_pt = OUTW // NTILE

        def zero_out(j, _):
            pltpu.sync_copy(
                zero_v,
                out_hbm.at[pl.ds(c * OUTW + s * cells_pt + j * 8256, 8256)])
            return _

        lax.fori_loop(0, cells_pt // 8256, zero_out, 0)

        plsc.subcore_barrier()

        def chunk(i, _):
            pltpu.sync_copy(tgt_hbm.at[c, s, pl.ds(i * CHUNK, CHUNK)], stg_v)
            ds = [
                pltpu.async_copy(ones_v, out_hbm.at[stg_v.at[j]], sem)
                for j in range(CHUNK)
            ]
            for d in ds:
                d.wait()
            return _

        lax.fori_loop(0, NCHUNK, chunk, 0)

    return k(tgt)


def _asm_body(sc_ref, g0r, g1r, g2r, g3r, out_ref):
    r = pl.program_id(0)
    g0 = g0r[0]
    g1 = g1r[0]
    g2 = g2r[0]
    g3 = g3r[0]
    avg = sc_ref[0, 0]
    step = sc_ref[0, 1]
    hx = sc_ref[0, 2]
    hy = sc_ref[0, 3]
    ii = lax.broadcasted_iota(jnp.int32, (BR, G), 0).astype(jnp.float32)
    jj = lax.broadcasted_iota(jnp.int32, (BR, G), 1).astype(jnp.float32)
    ccy = (r * BR + ii + 0.5) * (1.0 / G)
    ccx = (jj + 0.5) * (1.0 / G)
    one = jnp.ones((BR, G), jnp.float32)
    out_ref[0, 0] = g0
    out_ref[0, 1] = TANH1 * g1
    out_ref[0, 2] = TANH1 * g2
    out_ref[0, 3] = TANH1 * g3
    out_ref[0, 4] = g1
    out_ref[0, 5] = g2
    out_ref[0, 6] = g3
    out_ref[0, 7] = jnp.maximum(jnp.maximum(g1, g2), g3)
    out_ref[0, 8] = one
    out_ref[0, 9] = one
    out_ref[0, 10] = (hx - ccx) * VSCALE
    out_ref[0, 11] = (hy - ccy) * VSCALE
    out_ref[0, 12] = one * step
    out_ref[0, 13] = one * avg


def _assemble(grid2, consts):
    nrb = G // BR
    gspec = lambda h, off: pl.BlockSpec((1, BR, G), lambda r: (h, off + r, 0))
    return pl.pallas_call(
        _asm_body,
        grid=(nrb,),
        in_specs=[
            pl.BlockSpec((1, 4), lambda r: (0, 0), memory_space=pltpu.SMEM),
            gspec(0, 0), gspec(0, nrb), gspec(1, 0), gspec(1, nrb),
        ],
        out_specs=pl.BlockSpec((1, 14, BR, G), lambda r: (0, 0, r, 0)),
        out_shape=jax.ShapeDtypeStruct((1, 14, G, G), jnp.float32),
    )(consts, grid2, grid2, grid2, grid2)


def kernel(all_pos, all_types, all_teams, all_feat, own_hive_pos, team_id,
           current_step, grid_size, max_steps):
    px = all_pos[:, 0].reshape(NB, 1, B)
    py = all_pos[:, 1].reshape(NB, 1, B)
    ty = all_types.astype(jnp.int32).reshape(NB, 1, B)
    tm = all_teams.astype(jnp.int32).reshape(NB, 1, B)
    f0 = all_feat[:, 0].reshape(NB, 1, B)
    team = jnp.asarray(team_id, jnp.int32).reshape(1, 1)

    t0, t1, es, ec = _prep(px, py, ty, tm, f0, team)

    pad = jnp.full((NPAD - N,), DUMP, jnp.int32)
    t0f = jnp.concatenate([t0.reshape(-1), pad]).reshape(NTILE, ROWS_PT, 128)
    t1f = jnp.concatenate([t1.reshape(-1), pad]).reshape(NTILE, ROWS_PT, 128)
    tgt = jnp.stack([t0f, t1f])

    out2 = _sc_scatter(tgt)
    grid2 = out2.reshape(2, OUTW // G, G)

    cnt = ec[0, 0]
    avg = jnp.where(cnt > 0, es[0, 0] / jnp.maximum(cnt, 1.0), 0.0)
    step_frac = jnp.asarray(current_step, jnp.float32) / jnp.asarray(max_steps, jnp.float32)
    consts = jnp.stack([
        avg, step_frac, own_hive_pos[0], own_hive_pos[1]
    ]).reshape(1, 4).astype(jnp.float32)

    return _assemble(grid2, consts)


# speedup vs baseline: 32.2735x; 32.2735x over previous
import functools
import math

import jax
import jax.numpy as jnp
from jax import lax
from jax.experimental import pallas as pl
from jax.experimental.pallas import tpu as pltpu
from jax.experimental.pallas import tpu_sc as plsc

N = 2000000
G = 1024
GG = G * G
HALF = 2 * GG
PAD = 16384
OUTW = HALF + PAD
DUMP = HALF
GDUMP = 4 * GG

B = 16000
NB = N // B

NTILE = 16
ROWS_PT = 984
NPAD = NTILE * ROWS_PT * 128
CHUNK = 24
NCHUNK = ROWS_PT // CHUNK

BR = 64
TANH1 = math.tanh(1.0)
VSCALE = 2.0 / math.sqrt(2.0)


def _prep_body(team_ref, px, py, ty, tm, f0, t0_ref, t1_ref, es_ref, ec_ref):
    i = pl.program_id(0)
    x = px[0, 0, :]
    y = py[0, 0, :]
    t = ty[0, 0, :]
    m = tm[0, 0, :]
    team = team_ref[0, 0]
    mx = jnp.clip((x * float(G)).astype(jnp.int32), 0, G - 1)
    my = jnp.clip((y * float(G)).astype(jnp.int32), 0, G - 1)
    flat = my * G + mx
    is_obstacle = t == 3
    is_ally = (t == 0) & (m == team)
    is_enemy = (t == 0) & (m != team) & (m >= 0)
    is_resource = t == 1
    ch = jnp.where(
        is_obstacle, 0,
        jnp.where(is_ally, 1, jnp.where(is_enemy, 2, jnp.where(is_resource, 3, -1))))
    valid = ch >= 0
    flat4 = jnp.where(valid, flat + GG * jnp.maximum(ch, 0), GDUMP)
    dump0 = DUMP + (flat & (PAD - 1))
    t0 = jnp.where(flat4 < HALF, flat4, dump0)
    t1 = jnp.where((flat4 >= HALF) & (flat4 < 4 * GG),
                   flat4 - HALF + OUTW, dump0 + OUTW)
    t0_ref[0, 0, :] = t0
    t1_ref[0, 0, :] = t1
    ally_f = is_ally.astype(jnp.float32)
    bs = jnp.sum(f0[0, 0, :] * ally_f)
    bc = jnp.sum(ally_f)

    @pl.when(i == 0)
    def _():
        es_ref[0, 0] = 0.0
        ec_ref[0, 0] = 0.0

    es_ref[0, 0] += bs
    ec_ref[0, 0] += bc


def _prep(px, py, ty, tm, f0, team):
    spec = lambda dt: pl.BlockSpec((1, 1, B), lambda i: (i, 0, 0))
    sspec = pl.BlockSpec((1, 1), lambda i: (0, 0), memory_space=pltpu.SMEM)
    return pl.pallas_call(
        _prep_body,
        grid=(NB,),
        in_specs=[sspec, spec(None), spec(None), spec(None), spec(None), spec(None)],
        out_specs=[
            pl.BlockSpec((1, 1, B), lambda i: (i, 0, 0)),
            pl.BlockSpec((1, 1, B), lambda i: (i, 0, 0)),
            sspec, sspec,
        ],
        out_shape=[
            jax.ShapeDtypeStruct((NB, 1, B), jnp.int32),
            jax.ShapeDtypeStruct((NB, 1, B), jnp.int32),
            jax.ShapeDtypeStruct((1, 1), jnp.float32),
            jax.ShapeDtypeStruct((1, 1), jnp.float32),
        ],
    )(team, px, py, ty, tm, f0)


def _sc_scatter(tgt):
    mesh = plsc.VectorSubcoreMesh(core_axis_name="c", subcore_axis_name="s")

    @functools.partial(
        pl.kernel,
        mesh=mesh,
        out_type=jax.ShapeDtypeStruct((2 * OUTW,), jnp.float32),
        scratch_types=[
            pltpu.VMEM((CHUNK, 128), jnp.int32),
            pltpu.VMEM((128,), jnp.float32),
            pltpu.VMEM((8256,), jnp.float32),
            pltpu.SemaphoreType.DMA,
        ],
    )
    def k(tgt_hbm, out_hbm, stg_v, ones_v, zero_v, sem):
        c = lax.axis_index("c")
        s = lax.axis_index("s")

        def fill_ones(j, _):
            ones_v[pl.ds(j * 16, 16)] = jnp.ones((16,), jnp.float32)
            return _

        lax.fori_loop(0, 8, fill_ones, 0, unroll=True)

        def fill_zero(j, _):
            zero_v[pl.ds(j * 16, 16)] = jnp.zeros((16,), jnp.float32)
            return _

        lax.fori_loop(0, 516, fill_zero, 0)

        cells_pt = OUTW // NTILE

        def zero_out(j, _):
            pltpu.sync_copy(
                zero_v,
                out_hbm.at[pl.ds(c * OUTW + s * cells_pt + j * 8256, 8256)])
            return _

        lax.fori_loop(0, cells_pt // 8256, zero_out, 0)

        plsc.subcore_barrier()

        def chunk(i, _):
            pltpu.sync_copy(tgt_hbm.at[c, s, pl.ds(i * CHUNK, CHUNK)], stg_v)
            ds = [
                pltpu.async_copy(ones_v, out_hbm.at[stg_v.at[j]], sem)
                for j in range(CHUNK)
            ]
            for d in ds:
                d.wait()
            return _

        lax.fori_loop(0, NCHUNK, chunk, 0)

    return k(tgt)


def _asm_body(sc_ref, g0r, g1r, g2r, g3r, out_ref):
    r = pl.program_id(0)
    g0 = g0r[0]
    g1 = g1r[0]
    g2 = g2r[0]
    g3 = g3r[0]
    avg = sc_ref[0, 0]
    step = sc_ref[0, 1]
    hx = sc_ref[0, 2]
    hy = sc_ref[0, 3]
    ii = lax.broadcasted_iota(jnp.int32, (BR, G), 0).astype(jnp.float32)
    jj = lax.broadcasted_iota(jnp.int32, (BR, G), 1).astype(jnp.float32)
    ccy = (r * BR + ii + 0.5) * (1.0 / G)
    ccx = (jj + 0.5) * (1.0 / G)
    one = jnp.ones((BR, G), jnp.float32)
    out_ref[0, 0] = g0
    out_ref[0, 1] = TANH1 * g1
    out_ref[0, 2] = TANH1 * g2
    out_ref[0, 3] = TANH1 * g3
    out_ref[0, 4] = g1
    out_ref[0, 5] = g2
    out_ref[0, 6] = g3
    out_ref[0, 7] = jnp.maximum(jnp.maximum(g1, g2), g3)
    out_ref[0, 8] = one
    out_ref[0, 9] = one
    out_ref[0, 10] = (hx - ccx) * VSCALE
    out_ref[0, 11] = (hy - ccy) * VSCALE
    out_ref[0, 12] = one * step
    out_ref[0, 13] = one * avg


def _assemble(grid2, consts):
    nrb = G // BR
    gspec = lambda h, off: pl.BlockSpec((1, BR, G), lambda r: (h, off + r, 0))
    return pl.pallas_call(
        _asm_body,
        grid=(nrb,),
        in_specs=[
            pl.BlockSpec((1, 4), lambda r: (0, 0), memory_space=pltpu.SMEM),
            gspec(0, 0), gspec(0, nrb), gspec(1, 0), gspec(1, nrb),
        ],
        out_specs=pl.BlockSpec((1, 14, BR, G), lambda r: (0, 0, r, 0)),
        out_shape=jax.ShapeDtypeStruct((1, 14, G, G), jnp.float32),
    )(consts, grid2, grid2, grid2, grid2)


def kernel(all_pos, all_types, all_teams, all_feat, own_hive_pos, team_id,
           current_step, grid_size, max_steps):
    px = all_pos[:, 0].reshape(NB, 1, B)
    py = all_pos[:, 1].reshape(NB, 1, B)
    ty = all_types.astype(jnp.int32).reshape(NB, 1, B)
    tm = all_teams.astype(jnp.int32).reshape(NB, 1, B)
    f0 = all_feat[:, 0].reshape(NB, 1, B)
    team = jnp.asarray(team_id, jnp.int32).reshape(1, 1)

    t0, t1, es, ec = _prep(px, py, ty, tm, f0, team)

    pad = DUMP + (jnp.arange(NPAD - N, dtype=jnp.int32) % PAD)
    t0f = jnp.concatenate([t0.reshape(-1), pad]).reshape(NTILE, ROWS_PT, 128)
    t1f = jnp.concatenate([t1.reshape(-1), pad]).reshape(NTILE, ROWS_PT, 128)
    tgt = jnp.stack([t0f, t1f])

    out2 = _sc_scatter(tgt)
    grid2 = out2.reshape(2, OUTW // G, G)

    cnt = ec[0, 0]
    avg = jnp.where(cnt > 0, es[0, 0] / jnp.maximum(cnt, 1.0), 0.0)
    step_frac = jnp.asarray(current_step, jnp.float32) / jnp.asarray(max_steps, jnp.float32)
    consts = jnp.stack([
        avg, step_frac, own_hive_pos[0], own_hive_pos[1]
    ]).reshape(1, 4).astype(jnp.float32)

    return _assemble(grid2, consts)


# speedup vs baseline: 36.0668x; 1.1175x over previous
import functools
import math

import jax
import jax.numpy as jnp
from jax import lax
from jax.experimental import pallas as pl
from jax.experimental.pallas import tpu as pltpu
from jax.experimental.pallas import tpu_sc as plsc

N = 2000000
G = 1024
GG = G * G
HALF = 2 * GG
PAD = 16384
OUTW = HALF + PAD
DUMP = HALF
GDUMP = 4 * GG

B = 16000
NB = N // B

NTILE = 16
ROWS_PT = 984
NPAD = NTILE * ROWS_PT * 128
CHUNK = 24
NCHUNK = ROWS_PT // CHUNK

BR = 64
TANH1 = math.tanh(1.0)
VSCALE = 2.0 / math.sqrt(2.0)


def _prep_body(team_ref, px, py, ty, tm, f0, t0_ref, t1_ref, es_ref, ec_ref):
    i = pl.program_id(0)
    x = px[0, 0, :]
    y = py[0, 0, :]
    t = ty[0, 0, :]
    m = tm[0, 0, :]
    team = team_ref[0, 0]
    mx = jnp.clip((x * float(G)).astype(jnp.int32), 0, G - 1)
    my = jnp.clip((y * float(G)).astype(jnp.int32), 0, G - 1)
    flat = my * G + mx
    is_obstacle = t == 3
    is_ally = (t == 0) & (m == team)
    is_enemy = (t == 0) & (m != team) & (m >= 0)
    is_resource = t == 1
    ch = jnp.where(
        is_obstacle, 0,
        jnp.where(is_ally, 1, jnp.where(is_enemy, 2, jnp.where(is_resource, 3, -1))))
    valid = ch >= 0
    flat4 = jnp.where(valid, flat + GG * jnp.maximum(ch, 0), GDUMP)
    dump0 = DUMP + (flat & (PAD - 1))
    t0 = jnp.where(flat4 < HALF, flat4, dump0)
    t1 = jnp.where((flat4 >= HALF) & (flat4 < 4 * GG),
                   flat4 - HALF + OUTW, dump0 + OUTW)
    t0_ref[0, 0, :] = t0
    t1_ref[0, 0, :] = t1
    ally_f = is_ally.astype(jnp.float32)
    bs = jnp.sum(f0[0, 0, :] * ally_f)
    bc = jnp.sum(ally_f)

    @pl.when(i == 0)
    def _():
        es_ref[0, 0] = 0.0
        ec_ref[0, 0] = 0.0

    es_ref[0, 0] += bs
    ec_ref[0, 0] += bc


def _prep(px, py, ty, tm, f0, team):
    spec = lambda dt: pl.BlockSpec((1, 1, B), lambda i: (i, 0, 0))
    sspec = pl.BlockSpec((1, 1), lambda i: (0, 0), memory_space=pltpu.SMEM)
    return pl.pallas_call(
        _prep_body,
        grid=(NB,),
        in_specs=[sspec, spec(None), spec(None), spec(None), spec(None), spec(None)],
        out_specs=[
            pl.BlockSpec((1, 1, B), lambda i: (i, 0, 0)),
            pl.BlockSpec((1, 1, B), lambda i: (i, 0, 0)),
            sspec, sspec,
        ],
        out_shape=[
            jax.ShapeDtypeStruct((NB, 1, B), jnp.int32),
            jax.ShapeDtypeStruct((NB, 1, B), jnp.int32),
            jax.ShapeDtypeStruct((1, 1), jnp.float32),
            jax.ShapeDtypeStruct((1, 1), jnp.float32),
        ],
    )(team, px, py, ty, tm, f0)


def _sc_scatter(tgt):
    mesh = plsc.VectorSubcoreMesh(core_axis_name="c", subcore_axis_name="s")

    @functools.partial(
        pl.kernel,
        mesh=mesh,
        out_type=jax.ShapeDtypeStruct((2 * OUTW,), jnp.float32),
        scratch_types=[
            pltpu.VMEM((CHUNK, 128), jnp.int32),
            pltpu.VMEM((128,), jnp.float32),
            pltpu.VMEM((8256,), jnp.float32),
            pltpu.SemaphoreType.DMA,
        ],
    )
    def k(tgt_hbm, out_hbm, stg_v, ones_v, zero_v, sem):
        c = lax.axis_index("c")
        s = lax.axis_index("s")

        def fill_ones(j, _):
            ones_v[pl.ds(j * 16, 16)] = jnp.ones((16,), jnp.float32)
            return _

        lax.fori_loop(0, 8, fill_ones, 0, unroll=True)

        def fill_zero(j, _):
            zero_v[pl.ds(j * 16, 16)] = jnp.zeros((16,), jnp.float32)
            return _

        lax.fori_loop(0, 516, fill_zero, 0)

        cells_pt = OUTW // NTILE

        def zero_out(j, _):
            pltpu.sync_copy(
                zero_v,
                out_hbm.at[pl.ds(c * OUTW + s * cells_pt + j * 8256, 8256)])
            return _

        lax.fori_loop(0, cells_pt // 8256, zero_out, 0)

        plsc.subcore_barrier()

        def chunk(i, _):
            pltpu.sync_copy(tgt_hbm.at[c, s, pl.ds(i * CHUNK, CHUNK)], stg_v)
            ds = [
                pltpu.async_copy(ones_v, out_hbm.at[stg_v.at[j]], sem)
                for j in range(CHUNK)
            ]
            for d in ds:
                d.wait()
            return _

        lax.fori_loop(0, NCHUNK, chunk, 0)

    return k(tgt)


def _asm_body(sc_ref, g0r, g1r, g2r, g3r, out_ref):
    r = pl.program_id(0)
    g0 = g0r[0]
    g1 = g1r[0]
    g2 = g2r[0]
    g3 = g3r[0]
    avg = sc_ref[0, 0]
    step = sc_ref[0, 1]
    hx = sc_ref[0, 2]
    hy = sc_ref[0, 3]
    ii = lax.broadcasted_iota(jnp.int32, (BR, G), 0).astype(jnp.float32)
    jj = lax.broadcasted_iota(jnp.int32, (BR, G), 1).astype(jnp.float32)
    ccy = (r * BR + ii + 0.5) * (1.0 / G)
    ccx = (jj + 0.5) * (1.0 / G)
    one = jnp.ones((BR, G), jnp.float32)
    out_ref[0, 0] = g0
    out_ref[0, 1] = TANH1 * g1
    out_ref[0, 2] = TANH1 * g2
    out_ref[0, 3] = TANH1 * g3
    out_ref[0, 4] = g1
    out_ref[0, 5] = g2
    out_ref[0, 6] = g3
    out_ref[0, 7] = jnp.maximum(jnp.maximum(g1, g2), g3)
    out_ref[0, 8] = one
    out_ref[0, 9] = one
    out_ref[0, 10] = (hx - ccx) * VSCALE
    out_ref[0, 11] = (hy - ccy) * VSCALE
    out_ref[0, 12] = one * step
    out_ref[0, 13] = one * avg


def _assemble(grid2, consts):
    nrb = G // BR
    gspec = lambda h, off: pl.BlockSpec((1, BR, G), lambda r: (h, off + r, 0))
    return pl.pallas_call(
        _asm_body,
        grid=(nrb,),
        in_specs=[
            pl.BlockSpec((1, 4), lambda r: (0, 0), memory_space=pltpu.SMEM),
            gspec(0, 0), gspec(0, nrb), gspec(1, 0), gspec(1, nrb),
        ],
        out_specs=pl.BlockSpec((1, 14, BR, G), lambda r: (0, 0, r, 0)),
        out_shape=jax.ShapeDtypeStruct((1, 14, G, G), jnp.float32),
    )(consts, grid2, grid2, grid2, grid2)


def kernel(all_pos, all_types, all_teams, all_feat, own_hive_pos, team_id,
           current_step, grid_size, max_steps):
    px = all_pos[:, 0].reshape(NB, 1, B)
    py = all_pos[:, 1].reshape(NB, 1, B)
    ty = all_types.astype(jnp.int32).reshape(NB, 1, B)
    tm = all_teams.astype(jnp.int32).reshape(NB, 1, B)
    f0 = all_feat[:, 0].reshape(NB, 1, B)
    team = jnp.asarray(team_id, jnp.int32).reshape(1, 1)

    t0, t1, es, ec = _prep(px, py, ty, tm, f0, team)

    pad = DUMP + (jnp.arange(NPAD - N, dtype=jnp.int32) % PAD)
    t0f = jnp.concatenate([t0.reshape(-1), pad]).reshape(NTILE, ROWS_PT, 128)
    t1f = jnp.concatenate([t1.reshape(-1), pad + OUTW]).reshape(NTILE, ROWS_PT, 128)
    tgt = jnp.stack([t0f, t1f])

    out2 = _sc_scatter(tgt)
    grid2 = out2.reshape(2, OUTW // G, G)

    cnt = ec[0, 0]
    avg = jnp.where(cnt > 0, es[0, 0] / jnp.maximum(cnt, 1.0), 0.0)
    step_frac = jnp.asarray(current_step, jnp.float32) / jnp.asarray(max_steps, jnp.float32)
    consts = jnp.stack([
        avg, step_frac, own_hive_pos[0], own_hive_pos[1]
    ]).reshape(1, 4).astype(jnp.float32)

    return _assemble(grid2, consts)
